# baseline (speedup 1.0000x reference)
import functools

import jax
import jax.numpy as jnp
from jax import lax
from jax.experimental import pallas as pl
from jax.experimental.pallas import tpu as pltpu
from jax.experimental.pallas import tpu_sc as plsc

N = 10000
D_IN = 128
D_H = 128
D_OUT = 64
E = 320000

NC = 2
NS = 16
NW = NC * NS
NPAD = 10112


def _seg_sum_sc(h, src_w, dst_w, zeros, d, nbuf, nc_pair):
    chunk = src_w.shape[2]
    nc0, nc1 = nc_pair
    assert nc0 % nbuf == 0 and nc1 % nbuf == 0
    assert min(nc0, nc1) >= 2 * nbuf
    zrows = NPAD // NS
    mesh = plsc.VectorSubcoreMesh(core_axis_name="c", subcore_axis_name="s")

    @functools.partial(
        pl.kernel,
        out_type=jax.ShapeDtypeStruct((NC, NPAD, d), jnp.float32),
        mesh=mesh,
        compiler_params=pltpu.CompilerParams(use_tc_tiling_on_sc=False),
        scratch_types=[
            pltpu.VMEM((max(nc0, nc1), chunk), jnp.int32),
            pltpu.VMEM((max(nc0, nc1), chunk), jnp.int32),
            pltpu.VMEM((nbuf, chunk, d), jnp.float32),
            pltpu.VMEM_SHARED((NPAD, d), jnp.float32),
            pltpu.SemaphoreType.DMA((nbuf,)),
            pltpu.SemaphoreType.DMA((nbuf,)),
        ],
    )
    def k(h_hbm, src_hbm, dst_hbm, z_hbm, out_hbm, src_v, dst_v, rows_v,
          acc_sh, sems, ssems):
        cid = lax.axis_index("c")
        sid = lax.axis_index("s")
        wid = cid * NS + sid
        pltpu.sync_copy(z_hbm.at[pl.ds(sid * zrows, zrows)],
                        acc_sh.at[pl.ds(sid * zrows, zrows)])
        pltpu.sync_copy(src_hbm.at[wid], src_v)
        pltpu.sync_copy(dst_hbm.at[wid], dst_v)
        plsc.subcore_barrier()

        M, L = nbuf, nbuf // 2
        def gather(j, b):
            pltpu.async_copy(h_hbm.at[src_v.at[j]], rows_v.at[b], sems.at[b])

        def wait_gather(j, b):
            pltpu.make_async_copy(h_hbm.at[src_v.at[j]], rows_v.at[b],
                                  sems.at[b]).wait()

        def scatter(j, b):
            pltpu.async_copy(rows_v.at[b], acc_sh.at[dst_v.at[j]],
                             ssems.at[b], add=True)

        def wait_scatter(j, b):
            pltpu.make_async_copy(rows_v.at[b], acc_sh.at[dst_v.at[j]],
                                  ssems.at[b]).wait()

        ngroups = jnp.where(cid == 0, nc0 // M, nc1 // M)
        for b in range(L):
            gather(b, b)
        for b in range(M):
            bl = (b + L) % M
            if b + L - M >= 0:
                wait_scatter(b + L - M, bl)
            gather(b + L, bl)
            wait_gather(b, b)
            scatter(b, b)

        def group(gi, carry):
            for b in range(M):
                j = gi * M + b
                bl = (b + L) % M
                wait_scatter(j + L - M, bl)
                gather(j + L, bl)
                wait_gather(j, b)
                scatter(j, b)
            return carry

        lax.fori_loop(1, ngroups - 1, group, 0, unroll=False)
        jt = (ngroups - 1) * M
        for b in range(M):
            j = jt + b
            bl = (b + L) % M
            wait_scatter(j + L - M, bl)
            if b < M - L:
                gather(j + L, bl)
            wait_gather(j, b)
            scatter(j, b)
        for i in range(M - L):
            c = jt + L + i
            wait_scatter(c, (L + i) % M)
        plsc.subcore_barrier()
        pltpu.sync_copy(acc_sh.at[pl.ds(sid * zrows, zrows)],
                        out_hbm.at[cid, pl.ds(sid * zrows, zrows)])

    return k(h, src_w, dst_w, zeros)


def _mm_body(x_ref, w_ref, o_ref):
    o_ref[...] = jnp.dot(x_ref[...], w_ref[...],
                         preferred_element_type=jnp.float32)


def _relu_mm_body(p_ref, w_ref, o_ref):
    g = jnp.maximum(p_ref[0] + p_ref[1], 0.0)
    o = jnp.dot(g, w_ref[...], preferred_element_type=jnp.float32)
    rows = lax.broadcasted_iota(jnp.int32, o.shape, 0)
    o_ref[...] = jnp.where(rows < N, o, 0.0)


def _log_softmax_body(q_ref, o_ref):
    s = q_ref[0] + q_ref[1]
    m = jnp.max(s, axis=1, keepdims=True)
    e = jnp.exp(s - m)
    o_ref[...] = (s - m) - jnp.log(jnp.sum(e, axis=1, keepdims=True))


def _edge_block(s_part, d_part, chunk, nbuf):
    e = s_part.shape[0]
    epw = -(-e // (NS * chunk * nbuf)) * chunk * nbuf
    nc = epw // chunk
    pad = NS * epw - e
    s_w = jnp.concatenate([s_part, jnp.full((pad,), N, jnp.int32)])
    d_w = jnp.concatenate([d_part, jnp.arange(pad, dtype=jnp.int32) % NPAD])
    return s_w.reshape(NS, nc, chunk), d_w.reshape(NS, nc, chunk), nc


def _chunked_edges(src, dst, chunk, nbuf, e0):
    s0, d0, nc0 = _edge_block(src[:e0], dst[:e0], chunk, nbuf)
    s1, d1, nc1 = _edge_block(src[e0:], dst[e0:], chunk, nbuf)
    nmax = max(nc0, nc1)
    s0 = jnp.pad(s0, ((0, 0), (0, nmax - nc0), (0, 0)))
    d0 = jnp.pad(d0, ((0, 0), (0, nmax - nc0), (0, 0)))
    s1 = jnp.pad(s1, ((0, 0), (0, nmax - nc1), (0, 0)))
    d1 = jnp.pad(d1, ((0, 0), (0, nmax - nc1), (0, 0)))
    src_w = jnp.concatenate([s0, s1], axis=0)
    dst_w = jnp.concatenate([d0, d1], axis=0)
    return src_w, dst_w, (nc0, nc1)


def kernel(x, edge_index, W1, W2):
    src = edge_index[0].astype(jnp.int32)
    dst = edge_index[1].astype(jnp.int32)
    e0 = E // 2
    src1, dst1, ncp1 = _chunked_edges(src, dst, 64, 3, e0)
    src2, dst2, ncp2 = _chunked_edges(src, dst, 128, 4, e0)

    z_h = jnp.zeros((NPAD, D_H), jnp.float32)
    z_o = jnp.zeros((NPAD, D_OUT), jnp.float32)

    x_pad = jnp.concatenate([x, jnp.zeros((8, D_IN), jnp.float32)])
    h = pl.pallas_call(
        _mm_body,
        out_shape=jax.ShapeDtypeStruct((N + 8, D_H), jnp.float32),
    )(x_pad, W1)
    p = _seg_sum_sc(h, src1, dst1, z_h, D_H, nbuf=3, nc_pair=ncp1)

    h2 = pl.pallas_call(
        _relu_mm_body,
        out_shape=jax.ShapeDtypeStruct((NPAD, D_OUT), jnp.float32),
    )(p, W2)
    q = _seg_sum_sc(h2, src2, dst2, z_o, D_OUT, nbuf=4, nc_pair=ncp2)

    out = pl.pallas_call(
        _log_softmax_body,
        out_shape=jax.ShapeDtypeStruct((NPAD, D_OUT), jnp.float32),
    )(q)
    return out[:N]


# speedup vs baseline: 1.1321x; 1.1321x over previous
import functools

import jax
import jax.numpy as jnp
from jax import lax
from jax.experimental import pallas as pl
from jax.experimental.pallas import tpu as pltpu
from jax.experimental.pallas import tpu_sc as plsc

N = 10000
D_IN = 128
D_H = 128
D_OUT = 64
E = 320000

NC = 2
NS = 16
NW = NC * NS
NPAD = 10112


def _seg_sum_sc(h, src_w, dst_w, zeros, d, nbuf, nc_pair):
    chunk = src_w.shape[2]
    nc0, nc1 = nc_pair
    assert nc0 % nbuf == 0 and nc1 % nbuf == 0
    assert min(nc0, nc1) >= 2 * nbuf
    zrows = NPAD // NS
    mesh = plsc.VectorSubcoreMesh(core_axis_name="c", subcore_axis_name="s")

    @functools.partial(
        pl.kernel,
        out_type=jax.ShapeDtypeStruct((NC, NPAD, d), jnp.float32),
        mesh=mesh,
        compiler_params=pltpu.CompilerParams(use_tc_tiling_on_sc=False),
        scratch_types=[
            pltpu.VMEM((max(nc0, nc1), chunk), jnp.int32),
            pltpu.VMEM((max(nc0, nc1), chunk), jnp.int32),
            pltpu.VMEM((nbuf, chunk, d), jnp.float32),
            pltpu.VMEM_SHARED((NPAD, d), jnp.float32),
            pltpu.SemaphoreType.DMA((nbuf,)),
        ],
    )
    def k(h_hbm, src_hbm, dst_hbm, z_hbm, out_hbm, src_v, dst_v, rows_v,
          acc_sh, sems):
        cid = lax.axis_index("c")
        sid = lax.axis_index("s")
        wid = cid * NS + sid
        pltpu.sync_copy(z_hbm.at[pl.ds(sid * zrows, zrows)],
                        acc_sh.at[pl.ds(sid * zrows, zrows)])
        pltpu.sync_copy(src_hbm.at[wid], src_v)
        pltpu.sync_copy(dst_hbm.at[wid], dst_v)
        plsc.subcore_barrier()

        def gather(j, b):
            pltpu.async_copy(h_hbm.at[src_v.at[j]], rows_v.at[b], sems.at[b])

        def consume(j, b):
            pltpu.make_async_copy(h_hbm.at[src_v.at[j]], rows_v.at[b],
                                  sems.at[b]).wait()
            pltpu.sync_copy(rows_v.at[b], acc_sh.at[dst_v.at[j]], add=True)

        ngroups = jnp.where(cid == 0, nc0 // nbuf, nc1 // nbuf)
        for b in range(nbuf):
            gather(b, b)

        def group(gi, carry):
            for b in range(nbuf):
                j = gi * nbuf + b
                consume(j, b)
                gather(j + nbuf, b)
            return carry

        lax.fori_loop(0, ngroups - 1, group, 0, unroll=False)
        for b in range(nbuf):
            consume((ngroups - 1) * nbuf + b, b)
        plsc.subcore_barrier()
        pltpu.sync_copy(acc_sh.at[pl.ds(sid * zrows, zrows)],
                        out_hbm.at[cid, pl.ds(sid * zrows, zrows)])

    return k(h, src_w, dst_w, zeros)


def _mm_body(x_ref, w_ref, o_ref):
    o_ref[...] = jnp.dot(x_ref[...], w_ref[...],
                         preferred_element_type=jnp.float32)


def _relu_mm_body(p_ref, w_ref, o_ref):
    g = jnp.maximum(p_ref[0] + p_ref[1], 0.0)
    o = jnp.dot(g, w_ref[...], preferred_element_type=jnp.float32)
    rows = lax.broadcasted_iota(jnp.int32, o.shape, 0)
    o_ref[...] = jnp.where(rows < N, o, 0.0)


def _log_softmax_body(q_ref, o_ref):
    s = q_ref[0] + q_ref[1]
    m = jnp.max(s, axis=1, keepdims=True)
    e = jnp.exp(s - m)
    o_ref[...] = (s - m) - jnp.log(jnp.sum(e, axis=1, keepdims=True))


def _edge_block(s_part, d_part, chunk, nbuf):
    e = s_part.shape[0]
    epw = -(-e // (NS * chunk * nbuf)) * chunk * nbuf
    nc = epw // chunk
    pad = NS * epw - e
    s_w = jnp.concatenate([s_part, jnp.full((pad,), N, jnp.int32)])
    d_w = jnp.concatenate([d_part, jnp.arange(pad, dtype=jnp.int32) % NPAD])
    return s_w.reshape(NS, nc, chunk), d_w.reshape(NS, nc, chunk), nc


def _chunked_edges(src, dst, chunk, nbuf, e0):
    s0, d0, nc0 = _edge_block(src[:e0], dst[:e0], chunk, nbuf)
    s1, d1, nc1 = _edge_block(src[e0:], dst[e0:], chunk, nbuf)
    nmax = max(nc0, nc1)
    s0 = jnp.pad(s0, ((0, 0), (0, nmax - nc0), (0, 0)))
    d0 = jnp.pad(d0, ((0, 0), (0, nmax - nc0), (0, 0)))
    s1 = jnp.pad(s1, ((0, 0), (0, nmax - nc1), (0, 0)))
    d1 = jnp.pad(d1, ((0, 0), (0, nmax - nc1), (0, 0)))
    src_w = jnp.concatenate([s0, s1], axis=0)
    dst_w = jnp.concatenate([d0, d1], axis=0)
    return src_w, dst_w, (nc0, nc1)


def kernel(x, edge_index, W1, W2):
    src = edge_index[0].astype(jnp.int32)
    dst = edge_index[1].astype(jnp.int32)
    src1, dst1, ncp1 = _chunked_edges(src, dst, 64, 2, 92160)
    src2, dst2, ncp2 = _chunked_edges(src, dst, 128, 4, 99840)

    z_h = jnp.zeros((NPAD, D_H), jnp.float32)
    z_o = jnp.zeros((NPAD, D_OUT), jnp.float32)

    x_pad = jnp.concatenate([x, jnp.zeros((8, D_IN), jnp.float32)])
    h = pl.pallas_call(
        _mm_body,
        out_shape=jax.ShapeDtypeStruct((N + 8, D_H), jnp.float32),
    )(x_pad, W1)
    p = _seg_sum_sc(h, src1, dst1, z_h, D_H, nbuf=2, nc_pair=ncp1)

    h2 = pl.pallas_call(
        _relu_mm_body,
        out_shape=jax.ShapeDtypeStruct((NPAD, D_OUT), jnp.float32),
    )(p, W2)
    q = _seg_sum_sc(h2, src2, dst2, z_o, D_OUT, nbuf=4, nc_pair=ncp2)

    out = pl.pallas_call(
        _log_softmax_body,
        out_shape=jax.ShapeDtypeStruct((NPAD, D_OUT), jnp.float32),
    )(q)
    return out[:N]
